# baseline (speedup 1.0000x reference)
import functools

import jax
import jax.numpy as jnp
from jax import lax
from jax.experimental import pallas as pl
from jax.experimental.pallas import tpu as pltpu
from jax.experimental.pallas import tpu_sc as plsc

R = 4096
C = 8192
NC = 2
NS = 16
L = 16
NW = NC * NS
ROWS_PER_W = R // NW
NCHAIN = ROWS_PER_W // L
CHUNK = 512
NCHUNK = C // CHUNK


def _cumsum_body(x_hbm, o_hbm, buf):
    cid = lax.axis_index("c")
    sid = lax.axis_index("s")
    wid = sid * NC + cid
    row0 = wid * ROWS_PER_W

    lane = lax.iota(jnp.int32, L)
    row_idx = [lane + j * L for j in range(NCHAIN)]

    def body(c, carries):
        col = jnp.full((L,), c, jnp.int32)
        vals = [plsc.load_gather(buf, [row_idx[j], col]) for j in range(NCHAIN)]
        new = [carries[j] + vals[j] for j in range(NCHAIN)]
        for j in range(NCHAIN):
            plsc.store_scatter(buf, [row_idx[j], col], new[j])
        return tuple(new)

    def outer(ch, carries):
        c0 = ch * CHUNK
        pltpu.sync_copy(x_hbm.at[pl.ds(row0, ROWS_PER_W), pl.ds(c0, CHUNK)], buf)
        carries = plsc.parallel_loop(0, CHUNK, carry=carries, unroll=8)(body)
        pltpu.sync_copy(buf, o_hbm.at[pl.ds(row0, ROWS_PER_W), pl.ds(c0, CHUNK)])
        return carries

    carries = tuple(jnp.zeros((L,), jnp.float32) for _ in range(NCHAIN))
    lax.fori_loop(0, NCHUNK, outer, carries)


def _make_kernel():
    mesh = plsc.VectorSubcoreMesh(core_axis_name="c", subcore_axis_name="s")
    return functools.partial(
        pl.kernel,
        mesh=mesh,
        out_type=jax.ShapeDtypeStruct((R, C), jnp.float32),
        scratch_types=[pltpu.VMEM((ROWS_PER_W, CHUNK), jnp.float32)],
        compiler_params=pltpu.CompilerParams(
            use_tc_tiling_on_sc=False, needs_layout_passes=False
        ),
    )(_cumsum_body)


_sc_cumsum = _make_kernel()


def kernel(x):
    return _sc_cumsum(x.astype(jnp.float32))


# speedup vs baseline: 1.1301x; 1.1301x over previous
import functools

import jax
import jax.numpy as jnp
from jax import lax
from jax.experimental import pallas as pl
from jax.experimental.pallas import tpu as pltpu
from jax.experimental.pallas import tpu_sc as plsc

R = 4096
C = 8192
NC = 2
NS = 16
L = 16
NW = NC * NS
ROWS_PER_W = R // NW
NCHAIN = ROWS_PER_W // L
CHUNK = 512
NCHUNK = C // CHUNK


def _cumsum_body(x_hbm, o_hbm, buf):
    cid = lax.axis_index("c")
    sid = lax.axis_index("s")
    wid = sid * NC + cid
    row0 = wid * ROWS_PER_W

    lane = lax.iota(jnp.int32, L)
    rowbase = [(lane + j * L) * CHUNK for j in range(NCHAIN)]
    zero = jnp.zeros((L,), jnp.int32)

    def body(c, st):
        accs, idxs = st
        vals = [plsc.load_gather(buf, [zero, idxs[j]]) for j in range(NCHAIN)]
        new_accs = [accs[j] + vals[j] for j in range(NCHAIN)]
        for j in range(NCHAIN):
            plsc.store_scatter(buf, [zero, idxs[j]], new_accs[j])
        new_idxs = [idxs[j] + 1 for j in range(NCHAIN)]
        return (tuple(new_accs), tuple(new_idxs))

    def outer(ch, carries):
        c0 = ch * CHUNK
        pltpu.sync_copy(x_hbm.at[pl.ds(row0, ROWS_PER_W), pl.ds(c0, CHUNK)], buf)
        st = (carries, tuple(rowbase))
        carries, _ = plsc.parallel_loop(0, CHUNK, carry=st, unroll=8)(body)
        pltpu.sync_copy(buf, o_hbm.at[pl.ds(row0, ROWS_PER_W), pl.ds(c0, CHUNK)])
        return carries

    carries = tuple(jnp.zeros((L,), jnp.float32) for _ in range(NCHAIN))
    lax.fori_loop(0, NCHUNK, outer, carries)


def _make_kernel():
    mesh = plsc.VectorSubcoreMesh(core_axis_name="c", subcore_axis_name="s")
    return functools.partial(
        pl.kernel,
        mesh=mesh,
        out_type=jax.ShapeDtypeStruct((R, C), jnp.float32),
        scratch_types=[pltpu.VMEM((ROWS_PER_W, CHUNK), jnp.float32)],
        compiler_params=pltpu.CompilerParams(
            use_tc_tiling_on_sc=False, needs_layout_passes=False
        ),
    )(_cumsum_body)


_sc_cumsum = _make_kernel()


def kernel(x):
    return _sc_cumsum(x.astype(jnp.float32))


# speedup vs baseline: 1.6020x; 1.4176x over previous
import functools

import jax
import jax.numpy as jnp
from jax import lax
from jax.experimental import pallas as pl
from jax.experimental.pallas import tpu as pltpu
from jax.experimental.pallas import tpu_sc as plsc

R = 4096
C = 8192
NC = 2
NS = 16
L = 16
NW = NC * NS
ROWS_PER_W = R // NW
NCHAIN = ROWS_PER_W // L
CHUNK = 512
NCHUNK = C // CHUNK


def _cumsum_body(x_hbm, o_hbm, buf):
    cid = lax.axis_index("c")
    sid = lax.axis_index("s")
    wid = sid * NC + cid
    row0 = wid * ROWS_PER_W

    lane = lax.iota(jnp.int32, L)
    rowbase = [(lane + j * L) * CHUNK for j in range(NCHAIN)]
    zero = jnp.zeros((L,), jnp.int32)

    def body(c, st):
        accs, idxs = st
        vals = [plsc.load_gather(buf, [zero, idxs[j]]) for j in range(NCHAIN)]
        new_accs = [accs[j] + vals[j] for j in range(NCHAIN)]
        for j in range(NCHAIN):
            plsc.store_scatter(buf, [zero, idxs[j]], new_accs[j])
        new_idxs = [idxs[j] + 1 for j in range(NCHAIN)]
        return (tuple(new_accs), tuple(new_idxs))

    def outer(ch, carries):
        c0 = ch * CHUNK
        pltpu.sync_copy(x_hbm.at[pl.ds(row0, ROWS_PER_W), pl.ds(c0, CHUNK)], buf)
        st = (carries, tuple(rowbase))
        carries, _ = plsc.parallel_loop(0, CHUNK, carry=st, unroll=2)(body)
        pltpu.sync_copy(buf, o_hbm.at[pl.ds(row0, ROWS_PER_W), pl.ds(c0, CHUNK)])
        return carries

    carries = tuple(jnp.zeros((L,), jnp.float32) for _ in range(NCHAIN))
    lax.fori_loop(0, NCHUNK, outer, carries)


def _make_kernel():
    mesh = plsc.VectorSubcoreMesh(core_axis_name="c", subcore_axis_name="s")
    return functools.partial(
        pl.kernel,
        mesh=mesh,
        out_type=jax.ShapeDtypeStruct((R, C), jnp.float32),
        scratch_types=[pltpu.VMEM((ROWS_PER_W, CHUNK), jnp.float32)],
        compiler_params=pltpu.CompilerParams(
            use_tc_tiling_on_sc=False, needs_layout_passes=False
        ),
    )(_cumsum_body)


_sc_cumsum = _make_kernel()


def kernel(x):
    return _sc_cumsum(x.astype(jnp.float32))


# speedup vs baseline: 3.0856x; 1.9261x over previous
import functools

import jax
import jax.numpy as jnp
from jax import lax
from jax.experimental import pallas as pl
from jax.experimental.pallas import tpu as pltpu
from jax.experimental.pallas import tpu_sc as plsc

R = 4096
C = 8192
NC = 2
NS = 16
L = 16
NW = NC * NS
ROWS_PER_W = R // NW
NCHAIN = ROWS_PER_W // L
CHUNK = 512
NCHUNK = C // CHUNK


def _cumsum_body(x_hbm, o_hbm, buf):
    cid = lax.axis_index("c")
    sid = lax.axis_index("s")
    wid = sid * NC + cid
    row0 = wid * ROWS_PER_W

    lane = lax.iota(jnp.int32, L)
    rows = [lane + j * L for j in range(NCHAIN)]

    def body(c, st):
        accs, col = st
        vals = [plsc.load_gather(buf, [rows[j], col]) for j in range(NCHAIN)]
        new_accs = [accs[j] + vals[j] for j in range(NCHAIN)]
        for j in range(NCHAIN):
            plsc.store_scatter(buf, [rows[j], col], new_accs[j])
        return (tuple(new_accs), col + 1)

    def outer(ch, carries):
        c0 = ch * CHUNK
        pltpu.sync_copy(
            x_hbm.at[pl.ds(row0, ROWS_PER_W), pl.ds(c0, CHUNK)],
            buf.at[:, pl.ds(0, CHUNK)],
        )
        st = (carries, jnp.zeros((L,), jnp.int32))
        carries, _ = plsc.parallel_loop(0, CHUNK, carry=st, unroll=2)(body)
        pltpu.sync_copy(
            buf.at[:, pl.ds(0, CHUNK)],
            o_hbm.at[pl.ds(row0, ROWS_PER_W), pl.ds(c0, CHUNK)],
        )
        return carries

    carries = tuple(jnp.zeros((L,), jnp.float32) for _ in range(NCHAIN))
    lax.fori_loop(0, NCHUNK, outer, carries)


def _make_kernel():
    mesh = plsc.VectorSubcoreMesh(core_axis_name="c", subcore_axis_name="s")
    return functools.partial(
        pl.kernel,
        mesh=mesh,
        out_type=jax.ShapeDtypeStruct((R, C), jnp.float32),
        scratch_types=[pltpu.VMEM((ROWS_PER_W, CHUNK + 1), jnp.float32)],
        compiler_params=pltpu.CompilerParams(
            use_tc_tiling_on_sc=False, needs_layout_passes=False
        ),
    )(_cumsum_body)


_sc_cumsum = _make_kernel()


def kernel(x):
    return _sc_cumsum(x.astype(jnp.float32))


# speedup vs baseline: 3.3370x; 1.0815x over previous
import functools

import jax
import jax.numpy as jnp
from jax import lax
from jax.experimental import pallas as pl
from jax.experimental.pallas import tpu as pltpu
from jax.experimental.pallas import tpu_sc as plsc

R = 4096
C = 8192
NC = 2
NS = 16
L = 16
NW = NC * NS
ROWS_PER_W = R // NW
NCHAIN = ROWS_PER_W // L
CHUNK = 128
PAD = 136
NCHUNK = C // CHUNK
NPAIR = NCHUNK // 2


def _cumsum_body(x_hbm, o_hbm, ia, ib, oa, ob, sia, sib, soa, sob):
    cid = lax.axis_index("c")
    sid = lax.axis_index("s")
    wid = sid * NC + cid
    row0 = wid * ROWS_PER_W

    lane = lax.iota(jnp.int32, L)
    rows = [lane + j * L for j in range(NCHAIN)]

    def in_copy(buf, sem, ch):
        return pltpu.make_async_copy(
            x_hbm.at[pl.ds(row0, ROWS_PER_W), pl.ds(ch * CHUNK, CHUNK)],
            buf.at[:, pl.ds(0, CHUNK)],
            sem,
        )

    def out_copy(buf, sem, ch):
        return pltpu.make_async_copy(
            buf.at[:, pl.ds(0, CHUNK)],
            o_hbm.at[pl.ds(row0, ROWS_PER_W), pl.ds(ch * CHUNK, CHUNK)],
            sem,
        )

    def compute(src, dst, accs):
        def body(c, st):
            accs, col = st
            vals = [plsc.load_gather(src, [rows[j], col]) for j in range(NCHAIN)]
            new = [accs[j] + vals[j] for j in range(NCHAIN)]
            for j in range(NCHAIN):
                plsc.store_scatter(dst, [rows[j], col], new[j])
            return (tuple(new), col + 1)

        st = (accs, jnp.zeros((L,), jnp.int32))
        accs, _ = plsc.parallel_loop(0, CHUNK, carry=st, unroll=2)(body)
        return accs

    in_copy(ia, sia, 0).start()

    def pair(i, accs):
        ka = 2 * i
        in_copy(ia, sia, ka).wait()
        in_copy(ib, sib, ka + 1).start()

        @pl.when(i > 0)
        def _():
            out_copy(oa, soa, ka - 2).wait()

        accs = compute(ia, oa, accs)
        out_copy(oa, soa, ka).start()

        in_copy(ib, sib, ka + 1).wait()

        @pl.when(i < NPAIR - 1)
        def _():
            in_copy(ia, sia, ka + 2).start()

        @pl.when(i > 0)
        def _():
            out_copy(ob, sob, ka - 1).wait()

        accs = compute(ib, ob, accs)
        out_copy(ob, sob, ka + 1).start()
        return accs

    accs = tuple(jnp.zeros((L,), jnp.float32) for _ in range(NCHAIN))
    lax.fori_loop(0, NPAIR, pair, accs)
    out_copy(oa, soa, NCHUNK - 2).wait()
    out_copy(ob, sob, NCHUNK - 1).wait()


def _make_kernel():
    mesh = plsc.VectorSubcoreMesh(core_axis_name="c", subcore_axis_name="s")
    return functools.partial(
        pl.kernel,
        mesh=mesh,
        out_type=jax.ShapeDtypeStruct((R, C), jnp.float32),
        scratch_types=[
            pltpu.VMEM((ROWS_PER_W, PAD), jnp.float32),
            pltpu.VMEM((ROWS_PER_W, PAD), jnp.float32),
            pltpu.VMEM((ROWS_PER_W, PAD), jnp.float32),
            pltpu.VMEM((ROWS_PER_W, PAD), jnp.float32),
            pltpu.SemaphoreType.DMA,
            pltpu.SemaphoreType.DMA,
            pltpu.SemaphoreType.DMA,
            pltpu.SemaphoreType.DMA,
        ],
        compiler_params=pltpu.CompilerParams(
            use_tc_tiling_on_sc=False, needs_layout_passes=False
        ),
    )(_cumsum_body)


_sc_cumsum = _make_kernel()


def kernel(x):
    return _sc_cumsum(x.astype(jnp.float32))
